# baseline (speedup 1.0000x reference)
import functools

import jax
import jax.numpy as jnp
from jax import lax
from jax.experimental import pallas as pl
from jax.experimental.pallas import tpu as pltpu
from jax.experimental.pallas import tpu_sc as plsc

N = 10000
D = 128
E = 320000

NC = 2
NS = 16
C = 80
GROUP = 2000
E_CORE = E // NC
E_TILE = E_CORE // NS
N_GROUPS = E_TILE // GROUP
CHUNKS_PER_GROUP = GROUP // C
ROWS_TILE = N // NS

CNT_W = 632
STRIPE = 632
_STRIPE_CHUNKS = [(0, 80), (80, 80), (160, 80), (240, 80), (320, 80),
                  (400, 80), (480, 80), (560, 72)]


def _zero_fill_2d(ref, rows):
  z = jnp.zeros((16,), jnp.float32)

  def body(r, carry):
    for kk in range(D // 16):
      ref[r, pl.ds(kk * 16, 16)] = z
    return carry

  lax.fori_loop(0, rows, body, 0)


def _zero_fill_1d(ref, n16):
  z = jnp.zeros((16,), jnp.float32)

  def body(k, carry):
    ref[pl.ds(k * 16, 16)] = z
    return carry

  lax.fori_loop(0, n16, body, 0)


NBUF = 4
NCHUNK = E_TILE // C


def _make_sc_agg(with_counts):
  out_type = [jax.ShapeDtypeStruct((NC, N, D), jnp.float32)]
  scratch = {
      "src_a": pltpu.VMEM((GROUP,), jnp.int32),
      "src_b": pltpu.VMEM((GROUP,), jnp.int32),
      "dst_a": pltpu.VMEM((GROUP,), jnp.int32),
      "dst_b": pltpu.VMEM((GROUP,), jnp.int32),
      "isem0": pltpu.SemaphoreType.DMA,
      "isem1": pltpu.SemaphoreType.DMA,
  }
  for b in range(NBUF):
    scratch[f"rows{b}"] = pltpu.VMEM((C, D), jnp.float32)
    scratch[f"dstv{b}"] = pltpu.VMEM((C,), jnp.int32)
    scratch[f"gsem{b}"] = pltpu.SemaphoreType.DMA
    scratch[f"ssem{b}"] = pltpu.SemaphoreType.DMA
  scratch["acc_sh"] = pltpu.VMEM_SHARED((N, D), jnp.float32)
  if with_counts:
    out_type.append(jax.ShapeDtypeStruct((NC * N,), jnp.float32))
    scratch["ones_v"] = pltpu.VMEM((C,), jnp.float32)
    scratch["zcnt"] = pltpu.VMEM((640,), jnp.float32)
    scratch["cnt_sh"] = pltpu.VMEM_SHARED((N,), jnp.float32)
    for b in range(NBUF):
      scratch[f"csem{b}"] = pltpu.SemaphoreType.DMA

  mesh = plsc.VectorSubcoreMesh(core_axis_name="c", subcore_axis_name="s")

  def body(src_hbm, dst_hbm, x_hbm, *outs, **scr):
    if with_counts:
      out_hbm, cnt_hbm = outs
      ones_v, zcnt, cnt_sh = scr["ones_v"], scr["zcnt"], scr["cnt_sh"]
      csems = [scr[f"csem{b}"] for b in range(NBUF)]
    else:
      (out_hbm,) = outs
    acc_sh = scr["acc_sh"]
    srcb = [scr["src_a"], scr["src_b"]]
    dstb = [scr["dst_a"], scr["dst_b"]]
    isems = [scr["isem0"], scr["isem1"]]
    rows = [scr[f"rows{b}"] for b in range(NBUF)]
    dstv = [scr[f"dstv{b}"] for b in range(NBUF)]
    gsems = [scr[f"gsem{b}"] for b in range(NBUF)]
    ssems = [scr[f"ssem{b}"] for b in range(NBUF)]

    c = lax.axis_index("c")
    s = lax.axis_index("s")

    _zero_fill_2d(rows[0], C)
    row0 = pl.multiple_of(ROWS_TILE * s - lax.rem(s, 8), 8)
    zd = [None, None]
    for k, (off, w) in enumerate(_STRIPE_CHUNKS):
      p = k % 2
      if zd[p] is not None:
        zd[p].wait()
      zd[p] = pltpu.async_copy(rows[0].at[pl.ds(0, w)],
                               acc_sh.at[pl.ds(row0 + off, w)], isems[p])
    if with_counts:
      _zero_fill_1d(zcnt, 640 // 16)
      for kk in range(C // 16):
        ones_v[pl.ds(kk * 16, 16)] = jnp.ones((16,), jnp.float32)
      cstart = pl.multiple_of(ROWS_TILE * s - lax.rem(s, 8), 8)
      pltpu.sync_copy(zcnt.at[pl.ds(0, CNT_W)],
                      cnt_sh.at[pl.ds(cstart, CNT_W)])
    for d in zd:
      if d is not None:
        d.wait()

    plsc.subcore_barrier()

    ebase = c * E_CORE + s * E_TILE

    pltpu.sync_copy(src_hbm.at[pl.ds(ebase, GROUP)], srcb[0])
    pltpu.sync_copy(dst_hbm.at[pl.ds(ebase, GROUP)], dstb[0])

    gd = [None] * NBUF
    sd = [None] * NBUF
    cd = [None] * NBUF
    idxd = {}

    def _issue_scatter(j):
      pb = j % NBUF
      gd[pb].wait()
      sd[pb] = pltpu.async_copy(rows[pb], acc_sh.at[dstv[pb]], ssems[pb],
                                add=True)
      if with_counts:
        cd[pb] = pltpu.async_copy(ones_v, cnt_sh.at[dstv[pb]], csems[pb],
                                  add=True)

    for j in range(NCHUNK):
      g, r = divmod(j, CHUNKS_PER_GROUP)
      p = g % 2
      if r == 1 and g + 1 < N_GROUPS:
        q = (g + 1) % 2
        nbase = ebase + (g + 1) * GROUP
        idxd[g + 1] = (
            pltpu.async_copy(src_hbm.at[pl.ds(nbase, GROUP)], srcb[q],
                             isems[q]),
            pltpu.async_copy(dst_hbm.at[pl.ds(nbase, GROUP)], dstb[q],
                             isems[q]))
      if r == 0 and g > 0:
        for d in idxd.pop(g):
          d.wait()
      b = j % NBUF
      if sd[b] is not None:
        sd[b].wait()
      if with_counts and cd[b] is not None:
        cd[b].wait()
      for kk in range(C // 16):
        dstv[b][pl.ds(kk * 16, 16)] = dstb[p][pl.ds(r * C + kk * 16, 16)]
      gd[b] = pltpu.async_copy(x_hbm.at[srcb[p].at[pl.ds(r * C, C)]], rows[b],
                               gsems[b])
      if j >= 1:
        _issue_scatter(j - 1)
    _issue_scatter(NCHUNK - 1)
    for b in range(NBUF):
      if sd[b] is not None:
        sd[b].wait()
      if with_counts and cd[b] is not None:
        cd[b].wait()

    plsc.subcore_barrier()

    outd = [None] * NBUF
    for k, (off, w) in enumerate(_STRIPE_CHUNKS):
      b = k % NBUF
      if outd[b] is not None:
        outd[b].wait()
      pltpu.sync_copy(acc_sh.at[pl.ds(row0 + off, w)], rows[b].at[pl.ds(0, w)])
      outd[b] = pltpu.async_copy(rows[b].at[pl.ds(0, w)],
                                 out_hbm.at[c, pl.ds(row0 + off, w)], gsems[b])
    for d in outd:
      if d is not None:
        d.wait()
    if with_counts:
      cstart = pl.multiple_of(ROWS_TILE * s - lax.rem(s, 8), 8)
      cobase = pl.multiple_of(c * N + cstart, 8)
      pltpu.sync_copy(cnt_sh.at[pl.ds(cstart, CNT_W)], zcnt.at[pl.ds(0, CNT_W)])
      pltpu.sync_copy(zcnt.at[pl.ds(0, CNT_W)], cnt_hbm.at[pl.ds(cobase, CNT_W)])

  return pl.kernel(body, out_type=out_type, mesh=mesh, scratch_types=scratch)


_sc_agg_counts = _make_sc_agg(True)
_sc_agg = _make_sc_agg(False)


R = 1000
NB = N // R


def _tc_a_body(p0, p1, c0, c1, xr, wl, bl, wr, hout, stats):
  i = pl.program_id(0)
  inv = 1.0 / jnp.maximum(c0[...] + c1[...], 1.0)
  a = (p0[...] + p1[...]) * inv
  z = lax.dot_general(a, wl[...], (((1,), (1,)), ((), ())),
                      preferred_element_type=jnp.float32)
  z = z + lax.dot_general(xr[...], wr[...], (((1,), (1,)), ((), ())),
                          preferred_element_type=jnp.float32)
  z = z + bl[...]
  h = jnp.maximum(z, 0.0)
  hout[...] = h
  st = jnp.concatenate([jnp.sum(h, axis=0, keepdims=True),
                        jnp.sum(h * h, axis=0, keepdims=True)], axis=0)

  @pl.when(i == 0)
  def _():
    stats[...] = st

  @pl.when(i != 0)
  def _():
    stats[...] = stats[...] + st


_tc_a = pl.pallas_call(
    _tc_a_body,
    grid=(NB,),
    in_specs=[
        pl.BlockSpec((R, D), lambda i: (i, 0)),
        pl.BlockSpec((R, D), lambda i: (i, 0)),
        pl.BlockSpec((R, 1), lambda i: (i, 0)),
        pl.BlockSpec((R, 1), lambda i: (i, 0)),
        pl.BlockSpec((R, D), lambda i: (i, 0)),
        pl.BlockSpec((D, D), lambda i: (0, 0)),
        pl.BlockSpec((1, D), lambda i: (0, 0)),
        pl.BlockSpec((D, D), lambda i: (0, 0)),
    ],
    out_specs=[
        pl.BlockSpec((R, D), lambda i: (i, 0)),
        pl.BlockSpec((2, D), lambda i: (0, 0)),
    ],
    out_shape=[
        jax.ShapeDtypeStruct((N, D), jnp.float32),
        jax.ShapeDtypeStruct((2, D), jnp.float32),
    ],
)


def _tc_b_body(h, stats, g, b, out):
  m = stats[0:1, :] * (1.0 / N)
  ex2 = stats[1:2, :] * (1.0 / N)
  v = ex2 - m * m
  out[...] = (h[...] - m) * lax.rsqrt(v + 1e-5) * g[...] + b[...]


_tc_b = pl.pallas_call(
    _tc_b_body,
    grid=(NB,),
    in_specs=[
        pl.BlockSpec((R, D), lambda i: (i, 0)),
        pl.BlockSpec((2, D), lambda i: (0, 0)),
        pl.BlockSpec((1, D), lambda i: (0, 0)),
        pl.BlockSpec((1, D), lambda i: (0, 0)),
    ],
    out_specs=pl.BlockSpec((R, D), lambda i: (i, 0)),
    out_shape=jax.ShapeDtypeStruct((N, D), jnp.float32),
)


def _tc_b3_body(h, stats, g, b, x1, x2, wa, wb, wc, blin, out):
  m = stats[0:1, :] * (1.0 / N)
  ex2 = stats[1:2, :] * (1.0 / N)
  v = ex2 - m * m
  x3 = (h[...] - m) * lax.rsqrt(v + 1e-5) * g[...] + b[...]
  acc = lax.dot_general(x1[...], wa[...], (((1,), (1,)), ((), ())),
                        preferred_element_type=jnp.float32)
  acc = acc + lax.dot_general(x2[...], wb[...], (((1,), (1,)), ((), ())),
                              preferred_element_type=jnp.float32)
  acc = acc + lax.dot_general(x3, wc[...], (((1,), (1,)), ((), ())),
                              preferred_element_type=jnp.float32)
  out[...] = jnp.maximum(acc + blin[...], 0.0)


_tc_b3 = pl.pallas_call(
    _tc_b3_body,
    grid=(NB,),
    in_specs=[
        pl.BlockSpec((R, D), lambda i: (i, 0)),
        pl.BlockSpec((2, D), lambda i: (0, 0)),
        pl.BlockSpec((1, D), lambda i: (0, 0)),
        pl.BlockSpec((1, D), lambda i: (0, 0)),
        pl.BlockSpec((R, D), lambda i: (i, 0)),
        pl.BlockSpec((R, D), lambda i: (i, 0)),
        pl.BlockSpec((D, D), lambda i: (0, 0)),
        pl.BlockSpec((D, D), lambda i: (0, 0)),
        pl.BlockSpec((D, D), lambda i: (0, 0)),
        pl.BlockSpec((1, D), lambda i: (0, 0)),
    ],
    out_specs=pl.BlockSpec((R, D), lambda i: (i, 0)),
    out_shape=jax.ShapeDtypeStruct((N, D), jnp.float32),
)


def kernel(x, edge_index, Wl1, bl1, Wr1, g1, b1, Wl2, bl2, Wr2, g2, b2,
           Wl3, bl3, Wr3, g3, b3, Wlin, blin):
  src = edge_index[0]
  dst = edge_index[1]

  p1, cnt = _sc_agg_counts(src, dst, x)
  cnt = cnt.reshape(NC, N)
  c0 = cnt[0].reshape(N, 1)
  c1 = cnt[1].reshape(N, 1)

  bl1r, g1r, b1r = bl1.reshape(1, D), g1.reshape(1, D), b1.reshape(1, D)
  bl2r, g2r, b2r = bl2.reshape(1, D), g2.reshape(1, D), b2.reshape(1, D)
  bl3r, g3r, b3r = bl3.reshape(1, D), g3.reshape(1, D), b3.reshape(1, D)
  blinr = blin.reshape(1, D)

  h1, st1 = _tc_a(p1[0], p1[1], c0, c1, x, Wl1, bl1r, Wr1)
  x1 = _tc_b(h1, st1, g1r, b1r)

  (p2,) = _sc_agg(src, dst, x1)
  h2, st2 = _tc_a(p2[0], p2[1], c0, c1, x1, Wl2, bl2r, Wr2)
  x2 = _tc_b(h2, st2, g2r, b2r)

  (p3,) = _sc_agg(src, dst, x2)
  h3, st3 = _tc_a(p3[0], p3[1], c0, c1, x2, Wl3, bl3r, Wr3)
  out = _tc_b3(h3, st3, g3r, b3r, x1, x2,
               Wlin[:, :D], Wlin[:, D:2 * D], Wlin[:, 2 * D:], blinr)
  return out


# speedup vs baseline: 1.1503x; 1.1503x over previous
import functools

import jax
import jax.numpy as jnp
from jax import lax
from jax.experimental import pallas as pl
from jax.experimental.pallas import tpu as pltpu
from jax.experimental.pallas import tpu_sc as plsc

N = 10000
D = 128
E = 320000

NC = 2
NS = 16
C = 80
GROUP = 2000
E_CORE = E // NC
E_TILE = E_CORE // NS
N_GROUPS = E_TILE // GROUP
CHUNKS_PER_GROUP = GROUP // C
ROWS_TILE = N // NS

CNT_W = 632
STRIPE = 632
_STRIPE_CHUNKS = [(0, 80), (80, 80), (160, 80), (240, 80), (320, 80),
                  (400, 80), (480, 80), (560, 72)]


def _zero_fill_2d(ref, rows):
  z = jnp.zeros((16,), jnp.float32)

  def body(r, carry):
    for kk in range(D // 16):
      ref[r, pl.ds(kk * 16, 16)] = z
    return carry

  lax.fori_loop(0, rows, body, 0)


def _zero_fill_1d(ref, n16):
  z = jnp.zeros((16,), jnp.float32)

  def body(k, carry):
    ref[pl.ds(k * 16, 16)] = z
    return carry

  lax.fori_loop(0, n16, body, 0)


NBUF = 3
NCHUNK = E_TILE // C


def _make_sc_agg(with_counts):
  out_type = [jax.ShapeDtypeStruct((NC, N, D), jnp.float32)]
  scratch = {
      "src_a": pltpu.VMEM((GROUP,), jnp.int32),
      "src_b": pltpu.VMEM((GROUP,), jnp.int32),
      "dst_a": pltpu.VMEM((GROUP,), jnp.int32),
      "dst_b": pltpu.VMEM((GROUP,), jnp.int32),
      "isem0": pltpu.SemaphoreType.DMA,
      "isem1": pltpu.SemaphoreType.DMA,
  }
  for b in range(NBUF):
    scratch[f"rows{b}"] = pltpu.VMEM((C, D), jnp.float32)
    scratch[f"dstv{b}"] = pltpu.VMEM((C,), jnp.int32)
    scratch[f"gsem{b}"] = pltpu.SemaphoreType.DMA
    scratch[f"ssem{b}"] = pltpu.SemaphoreType.DMA
  scratch["acc_sh"] = pltpu.VMEM_SHARED((N, D), jnp.float32)
  if with_counts:
    out_type.append(jax.ShapeDtypeStruct((NC * N,), jnp.float32))
    scratch["ones_v"] = pltpu.VMEM((C,), jnp.float32)
    scratch["zcnt"] = pltpu.VMEM((640,), jnp.float32)
    scratch["cnt_sh"] = pltpu.VMEM_SHARED((N,), jnp.float32)
    for b in range(NBUF):
      scratch[f"csem{b}"] = pltpu.SemaphoreType.DMA

  mesh = plsc.VectorSubcoreMesh(core_axis_name="c", subcore_axis_name="s")

  def body(src_hbm, dst_hbm, x_hbm, *outs, **scr):
    if with_counts:
      out_hbm, cnt_hbm = outs
      ones_v, zcnt, cnt_sh = scr["ones_v"], scr["zcnt"], scr["cnt_sh"]
      csems = [scr[f"csem{b}"] for b in range(NBUF)]
    else:
      (out_hbm,) = outs
    acc_sh = scr["acc_sh"]
    srcb = [scr["src_a"], scr["src_b"]]
    dstb = [scr["dst_a"], scr["dst_b"]]
    isems = [scr["isem0"], scr["isem1"]]
    rows = [scr[f"rows{b}"] for b in range(NBUF)]
    dstv = [scr[f"dstv{b}"] for b in range(NBUF)]
    gsems = [scr[f"gsem{b}"] for b in range(NBUF)]
    ssems = [scr[f"ssem{b}"] for b in range(NBUF)]

    c = lax.axis_index("c")
    s = lax.axis_index("s")

    _zero_fill_2d(rows[0], C)
    row0 = pl.multiple_of(ROWS_TILE * s - lax.rem(s, 8), 8)
    zd = [None, None]
    for k, (off, w) in enumerate(_STRIPE_CHUNKS):
      p = k % 2
      if zd[p] is not None:
        zd[p].wait()
      zd[p] = pltpu.async_copy(rows[0].at[pl.ds(0, w)],
                               acc_sh.at[pl.ds(row0 + off, w)], isems[p])
    if with_counts:
      _zero_fill_1d(zcnt, 640 // 16)
      for kk in range(C // 16):
        ones_v[pl.ds(kk * 16, 16)] = jnp.ones((16,), jnp.float32)
      cstart = pl.multiple_of(ROWS_TILE * s - lax.rem(s, 8), 8)
      pltpu.sync_copy(zcnt.at[pl.ds(0, CNT_W)],
                      cnt_sh.at[pl.ds(cstart, CNT_W)])
    for d in zd:
      if d is not None:
        d.wait()

    plsc.subcore_barrier()

    ebase = c * E_CORE + s * E_TILE

    pltpu.sync_copy(src_hbm.at[pl.ds(ebase, GROUP)], srcb[0])
    pltpu.sync_copy(dst_hbm.at[pl.ds(ebase, GROUP)], dstb[0])

    gd = [None] * NBUF
    sd = [None] * NBUF
    cd = [None] * NBUF
    idxd = {}

    def _issue_scatter(j):
      pb = j % NBUF
      gd[pb].wait()
      sd[pb] = pltpu.async_copy(rows[pb], acc_sh.at[dstv[pb]], ssems[pb],
                                add=True)
      if with_counts:
        cd[pb] = pltpu.async_copy(ones_v, cnt_sh.at[dstv[pb]], csems[pb],
                                  add=True)

    for j in range(NCHUNK):
      g, r = divmod(j, CHUNKS_PER_GROUP)
      p = g % 2
      if r == 1 and g + 1 < N_GROUPS:
        q = (g + 1) % 2
        nbase = ebase + (g + 1) * GROUP
        idxd[g + 1] = (
            pltpu.async_copy(src_hbm.at[pl.ds(nbase, GROUP)], srcb[q],
                             isems[q]),
            pltpu.async_copy(dst_hbm.at[pl.ds(nbase, GROUP)], dstb[q],
                             isems[q]))
      if r == 0 and g > 0:
        for d in idxd.pop(g):
          d.wait()
      b = j % NBUF
      if sd[b] is not None:
        sd[b].wait()
      if with_counts and cd[b] is not None:
        cd[b].wait()
      for kk in range(C // 16):
        dstv[b][pl.ds(kk * 16, 16)] = dstb[p][pl.ds(r * C + kk * 16, 16)]
      gd[b] = pltpu.async_copy(x_hbm.at[srcb[p].at[pl.ds(r * C, C)]], rows[b],
                               gsems[b])
      if j >= 1:
        _issue_scatter(j - 1)
    _issue_scatter(NCHUNK - 1)
    for b in range(NBUF):
      if sd[b] is not None:
        sd[b].wait()
      if with_counts and cd[b] is not None:
        cd[b].wait()

    plsc.subcore_barrier()

    outd = [None] * NBUF
    for k, (off, w) in enumerate(_STRIPE_CHUNKS):
      b = k % NBUF
      if outd[b] is not None:
        outd[b].wait()
      pltpu.sync_copy(acc_sh.at[pl.ds(row0 + off, w)], rows[b].at[pl.ds(0, w)])
      outd[b] = pltpu.async_copy(rows[b].at[pl.ds(0, w)],
                                 out_hbm.at[c, pl.ds(row0 + off, w)], gsems[b])
    for d in outd:
      if d is not None:
        d.wait()
    if with_counts:
      cstart = pl.multiple_of(ROWS_TILE * s - lax.rem(s, 8), 8)
      cobase = pl.multiple_of(c * N + cstart, 8)
      pltpu.sync_copy(cnt_sh.at[pl.ds(cstart, CNT_W)], zcnt.at[pl.ds(0, CNT_W)])
      pltpu.sync_copy(zcnt.at[pl.ds(0, CNT_W)], cnt_hbm.at[pl.ds(cobase, CNT_W)])

  return pl.kernel(body, out_type=out_type, mesh=mesh, scratch_types=scratch)


_sc_agg_counts = _make_sc_agg(True)
_sc_agg = _make_sc_agg(False)


R = 1000
NB = N // R


def _tc_a_body(p0, p1, c0, c1, xr, wl, bl, wr, hout, stats):
  i = pl.program_id(0)
  inv = 1.0 / jnp.maximum(c0[...] + c1[...], 1.0)
  a = (p0[...] + p1[...]) * inv
  z = lax.dot_general(a, wl[...], (((1,), (1,)), ((), ())),
                      preferred_element_type=jnp.float32)
  z = z + lax.dot_general(xr[...], wr[...], (((1,), (1,)), ((), ())),
                          preferred_element_type=jnp.float32)
  z = z + bl[...]
  h = jnp.maximum(z, 0.0)
  hout[...] = h
  st = jnp.concatenate([jnp.sum(h, axis=0, keepdims=True),
                        jnp.sum(h * h, axis=0, keepdims=True)], axis=0)

  @pl.when(i == 0)
  def _():
    stats[...] = st

  @pl.when(i != 0)
  def _():
    stats[...] = stats[...] + st


_tc_a = pl.pallas_call(
    _tc_a_body,
    grid=(NB,),
    in_specs=[
        pl.BlockSpec((R, D), lambda i: (i, 0)),
        pl.BlockSpec((R, D), lambda i: (i, 0)),
        pl.BlockSpec((R, 1), lambda i: (i, 0)),
        pl.BlockSpec((R, 1), lambda i: (i, 0)),
        pl.BlockSpec((R, D), lambda i: (i, 0)),
        pl.BlockSpec((D, D), lambda i: (0, 0)),
        pl.BlockSpec((1, D), lambda i: (0, 0)),
        pl.BlockSpec((D, D), lambda i: (0, 0)),
    ],
    out_specs=[
        pl.BlockSpec((R, D), lambda i: (i, 0)),
        pl.BlockSpec((2, D), lambda i: (0, 0)),
    ],
    out_shape=[
        jax.ShapeDtypeStruct((N, D), jnp.float32),
        jax.ShapeDtypeStruct((2, D), jnp.float32),
    ],
)


def _bn_affine(stats, g, b):
  m = stats[0:1, :] * (1.0 / N)
  ex2 = stats[1:2, :] * (1.0 / N)
  v = ex2 - m * m
  a = lax.rsqrt(v + 1e-5) * g[...]
  bb = b[...] - m * a
  return a, bb


def _tc_a2_body(p0, p1, c0, c1, hp, stp, gp, bp, wl, bl, wr, hout, stats):
  i = pl.program_id(0)
  a, bb = _bn_affine(stp, gp, bp)
  cnt = c0[...] + c1[...]
  inv = 1.0 / jnp.maximum(cnt, 1.0)
  mask = jnp.where(cnt > 0.0, 1.0, 0.0)
  agg = ((p0[...] + p1[...]) * inv) * a + bb * mask
  xprev = hp[...] * a + bb
  z = lax.dot_general(agg, wl[...], (((1,), (1,)), ((), ())),
                      preferred_element_type=jnp.float32)
  z = z + lax.dot_general(xprev, wr[...], (((1,), (1,)), ((), ())),
                          preferred_element_type=jnp.float32)
  z = z + bl[...]
  h = jnp.maximum(z, 0.0)
  hout[...] = h
  st = jnp.concatenate([jnp.sum(h, axis=0, keepdims=True),
                        jnp.sum(h * h, axis=0, keepdims=True)], axis=0)

  @pl.when(i == 0)
  def _():
    stats[...] = st

  @pl.when(i != 0)
  def _():
    stats[...] = stats[...] + st


_tc_a2 = pl.pallas_call(
    _tc_a2_body,
    grid=(NB,),
    in_specs=[
        pl.BlockSpec((R, D), lambda i: (i, 0)),
        pl.BlockSpec((R, D), lambda i: (i, 0)),
        pl.BlockSpec((R, 1), lambda i: (i, 0)),
        pl.BlockSpec((R, 1), lambda i: (i, 0)),
        pl.BlockSpec((R, D), lambda i: (i, 0)),
        pl.BlockSpec((2, D), lambda i: (0, 0)),
        pl.BlockSpec((1, D), lambda i: (0, 0)),
        pl.BlockSpec((1, D), lambda i: (0, 0)),
        pl.BlockSpec((D, D), lambda i: (0, 0)),
        pl.BlockSpec((1, D), lambda i: (0, 0)),
        pl.BlockSpec((D, D), lambda i: (0, 0)),
    ],
    out_specs=[
        pl.BlockSpec((R, D), lambda i: (i, 0)),
        pl.BlockSpec((2, D), lambda i: (0, 0)),
    ],
    out_shape=[
        jax.ShapeDtypeStruct((N, D), jnp.float32),
        jax.ShapeDtypeStruct((2, D), jnp.float32),
    ],
)


def _tc_head_body(h1, st1, g1, b1, h2, st2, g2, b2, h3, st3, g3, b3,
                  wa, wb, wc, blin, out):
  a1, bb1 = _bn_affine(st1, g1, b1)
  a2, bb2 = _bn_affine(st2, g2, b2)
  a3, bb3 = _bn_affine(st3, g3, b3)
  x1 = h1[...] * a1 + bb1
  x2 = h2[...] * a2 + bb2
  x3 = h3[...] * a3 + bb3
  acc = lax.dot_general(x1, wa[...], (((1,), (1,)), ((), ())),
                        preferred_element_type=jnp.float32)
  acc = acc + lax.dot_general(x2, wb[...], (((1,), (1,)), ((), ())),
                              preferred_element_type=jnp.float32)
  acc = acc + lax.dot_general(x3, wc[...], (((1,), (1,)), ((), ())),
                              preferred_element_type=jnp.float32)
  out[...] = jnp.maximum(acc + blin[...], 0.0)


_RD = pl.BlockSpec((R, D), lambda i: (i, 0))
_SD = pl.BlockSpec((2, D), lambda i: (0, 0))
_OD = pl.BlockSpec((1, D), lambda i: (0, 0))
_WD = pl.BlockSpec((D, D), lambda i: (0, 0))

_tc_head = pl.pallas_call(
    _tc_head_body,
    grid=(NB,),
    in_specs=[_RD, _SD, _OD, _OD, _RD, _SD, _OD, _OD, _RD, _SD, _OD, _OD,
              _WD, _WD, _WD, _OD],
    out_specs=pl.BlockSpec((R, D), lambda i: (i, 0)),
    out_shape=jax.ShapeDtypeStruct((N, D), jnp.float32),
)


def kernel(x, edge_index, Wl1, bl1, Wr1, g1, b1, Wl2, bl2, Wr2, g2, b2,
           Wl3, bl3, Wr3, g3, b3, Wlin, blin):
  src = edge_index[0]
  dst = edge_index[1]

  p1, cnt = _sc_agg_counts(src, dst, x)
  cnt = cnt.reshape(NC, N)
  c0 = cnt[0].reshape(N, 1)
  c1 = cnt[1].reshape(N, 1)

  bl1r, g1r, b1r = bl1.reshape(1, D), g1.reshape(1, D), b1.reshape(1, D)
  bl2r, g2r, b2r = bl2.reshape(1, D), g2.reshape(1, D), b2.reshape(1, D)
  bl3r, g3r, b3r = bl3.reshape(1, D), g3.reshape(1, D), b3.reshape(1, D)
  blinr = blin.reshape(1, D)

  h1, st1 = _tc_a(p1[0], p1[1], c0, c1, x, Wl1, bl1r, Wr1)

  (p2,) = _sc_agg(src, dst, h1)
  h2, st2 = _tc_a2(p2[0], p2[1], c0, c1, h1, st1, g1r, b1r, Wl2, bl2r, Wr2)

  (p3,) = _sc_agg(src, dst, h2)
  h3, st3 = _tc_a2(p3[0], p3[1], c0, c1, h2, st2, g2r, b2r, Wl3, bl3r, Wr3)

  out = _tc_head(h1, st1, g1r, b1r, h2, st2, g2r, b2r, h3, st3, g3r, b3r,
                 Wlin[:, :D], Wlin[:, D:2 * D], Wlin[:, 2 * D:], blinr)
  return out


# speedup vs baseline: 1.1560x; 1.0049x over previous
import functools

import jax
import jax.numpy as jnp
from jax import lax
from jax.experimental import pallas as pl
from jax.experimental.pallas import tpu as pltpu
from jax.experimental.pallas import tpu_sc as plsc

N = 10000
D = 128
E = 320000

NC = 2
NS = 16
C = 80
GROUP = 2000
E_CORE = E // NC
E_TILE = E_CORE // NS
N_GROUPS = E_TILE // GROUP
CHUNKS_PER_GROUP = GROUP // C
ROWS_TILE = N // NS

CNT_W = 632
STRIPE = 632
_STRIPE_CHUNKS = [(0, 80), (80, 80), (160, 80), (240, 80), (320, 80),
                  (400, 80), (480, 80), (560, 72)]


def _zero_fill_2d(ref, rows):
  z = jnp.zeros((16,), jnp.float32)

  def body(r, carry):
    for kk in range(D // 16):
      ref[r, pl.ds(kk * 16, 16)] = z
    return carry

  lax.fori_loop(0, rows, body, 0)


def _zero_fill_1d(ref, n16):
  z = jnp.zeros((16,), jnp.float32)

  def body(k, carry):
    ref[pl.ds(k * 16, 16)] = z
    return carry

  lax.fori_loop(0, n16, body, 0)


NBUF = 3
NCHUNK = E_TILE // C


def _make_sc_agg(with_counts):
  out_type = [jax.ShapeDtypeStruct((NC, N, D), jnp.float32)]
  scratch = {
      "src_a": pltpu.VMEM((GROUP,), jnp.int32),
      "src_b": pltpu.VMEM((GROUP,), jnp.int32),
      "dst_a": pltpu.VMEM((GROUP,), jnp.int32),
      "dst_b": pltpu.VMEM((GROUP,), jnp.int32),
      "isem0": pltpu.SemaphoreType.DMA,
      "isem1": pltpu.SemaphoreType.DMA,
  }
  for b in range(NBUF):
    scratch[f"rows{b}"] = pltpu.VMEM((C, D), jnp.float32)
    scratch[f"dstv{b}"] = pltpu.VMEM((C,), jnp.int32)
    scratch[f"gsem{b}"] = pltpu.SemaphoreType.DMA
    scratch[f"ssem{b}"] = pltpu.SemaphoreType.DMA
  scratch["acc_sh"] = pltpu.VMEM_SHARED((N, D), jnp.float32)
  if with_counts:
    out_type.append(jax.ShapeDtypeStruct((NC * N,), jnp.float32))
    scratch["ones_v"] = pltpu.VMEM((C,), jnp.float32)
    scratch["zcnt"] = pltpu.VMEM((640,), jnp.float32)
    scratch["cnt_sh"] = pltpu.VMEM_SHARED((N,), jnp.float32)
    for b in range(NBUF):
      scratch[f"csem{b}"] = pltpu.SemaphoreType.DMA

  mesh = plsc.VectorSubcoreMesh(core_axis_name="c", subcore_axis_name="s")

  def body(src_hbm, dst_hbm, x_hbm, *outs, **scr):
    if with_counts:
      out_hbm, cnt_hbm = outs
      ones_v, zcnt, cnt_sh = scr["ones_v"], scr["zcnt"], scr["cnt_sh"]
      csems = [scr[f"csem{b}"] for b in range(NBUF)]
    else:
      (out_hbm,) = outs
    acc_sh = scr["acc_sh"]
    srcb = [scr["src_a"], scr["src_b"]]
    dstb = [scr["dst_a"], scr["dst_b"]]
    isems = [scr["isem0"], scr["isem1"]]
    rows = [scr[f"rows{b}"] for b in range(NBUF)]
    dstv = [scr[f"dstv{b}"] for b in range(NBUF)]
    gsems = [scr[f"gsem{b}"] for b in range(NBUF)]
    ssems = [scr[f"ssem{b}"] for b in range(NBUF)]

    c = lax.axis_index("c")
    s = lax.axis_index("s")

    _zero_fill_2d(rows[NBUF - 1], C)
    row0 = pl.multiple_of(ROWS_TILE * s - lax.rem(s, 8), 8)
    zd = [None, None]
    for k, (off, w) in enumerate(_STRIPE_CHUNKS):
      p = k % 2
      if zd[p] is not None:
        zd[p].wait()
      zd[p] = pltpu.async_copy(rows[NBUF - 1].at[pl.ds(0, w)],
                               acc_sh.at[pl.ds(row0 + off, w)], isems[p])
    if with_counts:
      _zero_fill_1d(zcnt, 640 // 16)
      for kk in range(C // 16):
        ones_v[pl.ds(kk * 16, 16)] = jnp.ones((16,), jnp.float32)
      cstart = pl.multiple_of(ROWS_TILE * s - lax.rem(s, 8), 8)
      pltpu.sync_copy(zcnt.at[pl.ds(0, CNT_W)],
                      cnt_sh.at[pl.ds(cstart, CNT_W)])

    ebase = c * E_CORE + s * E_TILE

    pltpu.sync_copy(src_hbm.at[pl.ds(ebase, GROUP)], srcb[0])
    pltpu.sync_copy(dst_hbm.at[pl.ds(ebase, GROUP)], dstb[0])

    gd = [None] * NBUF
    sd = [None] * NBUF
    cd = [None] * NBUF
    idxd = {}

    def _issue_scatter(j):
      pb = j % NBUF
      gd[pb].wait()
      sd[pb] = pltpu.async_copy(rows[pb], acc_sh.at[dstv[pb]], ssems[pb],
                                add=True)
      if with_counts:
        cd[pb] = pltpu.async_copy(ones_v, cnt_sh.at[dstv[pb]], csems[pb],
                                  add=True)

    for j in range(NCHUNK):
      g, r = divmod(j, CHUNKS_PER_GROUP)
      p = g % 2
      if r == 1 and g + 1 < N_GROUPS:
        q = (g + 1) % 2
        nbase = ebase + (g + 1) * GROUP
        idxd[g + 1] = (
            pltpu.async_copy(src_hbm.at[pl.ds(nbase, GROUP)], srcb[q],
                             isems[q]),
            pltpu.async_copy(dst_hbm.at[pl.ds(nbase, GROUP)], dstb[q],
                             isems[q]))
      if r == 0 and g > 0:
        for d in idxd.pop(g):
          d.wait()
      if j == 1:
        for d in zd:
          if d is not None:
            d.wait()
        plsc.subcore_barrier()
      b = j % NBUF
      if sd[b] is not None:
        sd[b].wait()
      if with_counts and cd[b] is not None:
        cd[b].wait()
      for kk in range(C // 16):
        dstv[b][pl.ds(kk * 16, 16)] = dstb[p][pl.ds(r * C + kk * 16, 16)]
      gd[b] = pltpu.async_copy(x_hbm.at[srcb[p].at[pl.ds(r * C, C)]], rows[b],
                               gsems[b])
      if j >= 1:
        _issue_scatter(j - 1)
    _issue_scatter(NCHUNK - 1)
    for b in range(NBUF):
      if sd[b] is not None:
        sd[b].wait()
      if with_counts and cd[b] is not None:
        cd[b].wait()

    plsc.subcore_barrier()

    ind = [None] * NBUF
    outd = [None] * NBUF

    def _issue_out(k):
      pb = k % NBUF
      off, w = _STRIPE_CHUNKS[k]
      ind[pb].wait()
      outd[pb] = pltpu.async_copy(rows[pb].at[pl.ds(0, w)],
                                  out_hbm.at[c, pl.ds(row0 + off, w)],
                                  ssems[pb])

    for k, (off, w) in enumerate(_STRIPE_CHUNKS):
      b = k % NBUF
      if outd[b] is not None:
        outd[b].wait()
      ind[b] = pltpu.async_copy(acc_sh.at[pl.ds(row0 + off, w)],
                                rows[b].at[pl.ds(0, w)], gsems[b])
      if k >= 1:
        _issue_out(k - 1)
    _issue_out(len(_STRIPE_CHUNKS) - 1)
    for d in outd:
      if d is not None:
        d.wait()
    if with_counts:
      cstart = pl.multiple_of(ROWS_TILE * s - lax.rem(s, 8), 8)
      cobase = pl.multiple_of(c * N + cstart, 8)
      pltpu.sync_copy(cnt_sh.at[pl.ds(cstart, CNT_W)], zcnt.at[pl.ds(0, CNT_W)])
      pltpu.sync_copy(zcnt.at[pl.ds(0, CNT_W)], cnt_hbm.at[pl.ds(cobase, CNT_W)])

  return pl.kernel(body, out_type=out_type, mesh=mesh, scratch_types=scratch)


_sc_agg_counts = _make_sc_agg(True)
_sc_agg = _make_sc_agg(False)


R = 1000
NB = N // R


def _tc_a_body(p0, p1, c0, c1, xr, wl, bl, wr, hout, stats):
  i = pl.program_id(0)
  inv = 1.0 / jnp.maximum(c0[...] + c1[...], 1.0)
  a = (p0[...] + p1[...]) * inv
  z = lax.dot_general(a, wl[...], (((1,), (1,)), ((), ())),
                      preferred_element_type=jnp.float32)
  z = z + lax.dot_general(xr[...], wr[...], (((1,), (1,)), ((), ())),
                          preferred_element_type=jnp.float32)
  z = z + bl[...]
  h = jnp.maximum(z, 0.0)
  hout[...] = h
  st = jnp.concatenate([jnp.sum(h, axis=0, keepdims=True),
                        jnp.sum(h * h, axis=0, keepdims=True)], axis=0)

  @pl.when(i == 0)
  def _():
    stats[...] = st

  @pl.when(i != 0)
  def _():
    stats[...] = stats[...] + st


_tc_a = pl.pallas_call(
    _tc_a_body,
    grid=(NB,),
    in_specs=[
        pl.BlockSpec((R, D), lambda i: (i, 0)),
        pl.BlockSpec((R, D), lambda i: (i, 0)),
        pl.BlockSpec((R, 1), lambda i: (i, 0)),
        pl.BlockSpec((R, 1), lambda i: (i, 0)),
        pl.BlockSpec((R, D), lambda i: (i, 0)),
        pl.BlockSpec((D, D), lambda i: (0, 0)),
        pl.BlockSpec((1, D), lambda i: (0, 0)),
        pl.BlockSpec((D, D), lambda i: (0, 0)),
    ],
    out_specs=[
        pl.BlockSpec((R, D), lambda i: (i, 0)),
        pl.BlockSpec((2, D), lambda i: (0, 0)),
    ],
    out_shape=[
        jax.ShapeDtypeStruct((N, D), jnp.float32),
        jax.ShapeDtypeStruct((2, D), jnp.float32),
    ],
)


def _bn_affine(stats, g, b):
  m = stats[0:1, :] * (1.0 / N)
  ex2 = stats[1:2, :] * (1.0 / N)
  v = ex2 - m * m
  a = lax.rsqrt(v + 1e-5) * g[...]
  bb = b[...] - m * a
  return a, bb


def _tc_a2_body(p0, p1, c0, c1, hp, stp, gp, bp, wl, bl, wr, hout, stats):
  i = pl.program_id(0)
  a, bb = _bn_affine(stp, gp, bp)
  cnt = c0[...] + c1[...]
  inv = 1.0 / jnp.maximum(cnt, 1.0)
  mask = jnp.where(cnt > 0.0, 1.0, 0.0)
  agg = ((p0[...] + p1[...]) * inv) * a + bb * mask
  xprev = hp[...] * a + bb
  z = lax.dot_general(agg, wl[...], (((1,), (1,)), ((), ())),
                      preferred_element_type=jnp.float32)
  z = z + lax.dot_general(xprev, wr[...], (((1,), (1,)), ((), ())),
                          preferred_element_type=jnp.float32)
  z = z + bl[...]
  h = jnp.maximum(z, 0.0)
  hout[...] = h
  st = jnp.concatenate([jnp.sum(h, axis=0, keepdims=True),
                        jnp.sum(h * h, axis=0, keepdims=True)], axis=0)

  @pl.when(i == 0)
  def _():
    stats[...] = st

  @pl.when(i != 0)
  def _():
    stats[...] = stats[...] + st


_tc_a2 = pl.pallas_call(
    _tc_a2_body,
    grid=(NB,),
    in_specs=[
        pl.BlockSpec((R, D), lambda i: (i, 0)),
        pl.BlockSpec((R, D), lambda i: (i, 0)),
        pl.BlockSpec((R, 1), lambda i: (i, 0)),
        pl.BlockSpec((R, 1), lambda i: (i, 0)),
        pl.BlockSpec((R, D), lambda i: (i, 0)),
        pl.BlockSpec((2, D), lambda i: (0, 0)),
        pl.BlockSpec((1, D), lambda i: (0, 0)),
        pl.BlockSpec((1, D), lambda i: (0, 0)),
        pl.BlockSpec((D, D), lambda i: (0, 0)),
        pl.BlockSpec((1, D), lambda i: (0, 0)),
        pl.BlockSpec((D, D), lambda i: (0, 0)),
    ],
    out_specs=[
        pl.BlockSpec((R, D), lambda i: (i, 0)),
        pl.BlockSpec((2, D), lambda i: (0, 0)),
    ],
    out_shape=[
        jax.ShapeDtypeStruct((N, D), jnp.float32),
        jax.ShapeDtypeStruct((2, D), jnp.float32),
    ],
)


def _tc_head_body(h1, st1, g1, b1, h2, st2, g2, b2, h3, st3, g3, b3,
                  wlin, blin, out):
  a1, bb1 = _bn_affine(st1, g1, b1)
  a2, bb2 = _bn_affine(st2, g2, b2)
  a3, bb3 = _bn_affine(st3, g3, b3)
  x1 = h1[...] * a1 + bb1
  x2 = h2[...] * a2 + bb2
  x3 = h3[...] * a3 + bb3
  xc = jnp.concatenate([x1, x2, x3], axis=1)
  acc = lax.dot_general(xc, wlin[...], (((1,), (1,)), ((), ())),
                        preferred_element_type=jnp.float32)
  out[...] = jnp.maximum(acc + blin[...], 0.0)


_RD = pl.BlockSpec((R, D), lambda i: (i, 0))
_SD = pl.BlockSpec((2, D), lambda i: (0, 0))
_OD = pl.BlockSpec((1, D), lambda i: (0, 0))

_tc_head = pl.pallas_call(
    _tc_head_body,
    grid=(NB,),
    in_specs=[_RD, _SD, _OD, _OD, _RD, _SD, _OD, _OD, _RD, _SD, _OD, _OD,
              pl.BlockSpec((D, 3 * D), lambda i: (0, 0)), _OD],
    out_specs=pl.BlockSpec((R, D), lambda i: (i, 0)),
    out_shape=jax.ShapeDtypeStruct((N, D), jnp.float32),
)


def kernel(x, edge_index, Wl1, bl1, Wr1, g1, b1, Wl2, bl2, Wr2, g2, b2,
           Wl3, bl3, Wr3, g3, b3, Wlin, blin):
  src = edge_index[0]
  dst = edge_index[1]

  p1, cnt = _sc_agg_counts(src, dst, x)
  cnt = cnt.reshape(NC, N)
  c0 = cnt[0].reshape(N, 1)
  c1 = cnt[1].reshape(N, 1)

  bl1r, g1r, b1r = bl1.reshape(1, D), g1.reshape(1, D), b1.reshape(1, D)
  bl2r, g2r, b2r = bl2.reshape(1, D), g2.reshape(1, D), b2.reshape(1, D)
  bl3r, g3r, b3r = bl3.reshape(1, D), g3.reshape(1, D), b3.reshape(1, D)
  blinr = blin.reshape(1, D)

  h1, st1 = _tc_a(p1[0], p1[1], c0, c1, x, Wl1, bl1r, Wr1)

  (p2,) = _sc_agg(src, dst, h1)
  h2, st2 = _tc_a2(p2[0], p2[1], c0, c1, h1, st1, g1r, b1r, Wl2, bl2r, Wr2)

  (p3,) = _sc_agg(src, dst, h2)
  h3, st3 = _tc_a2(p3[0], p3[1], c0, c1, h2, st2, g2r, b2r, Wl3, bl3r, Wr3)

  out = _tc_head(h1, st1, g1r, b1r, h2, st2, g2r, b2r, h3, st3, g3r, b3r,
                 Wlin, blinr)
  return out
